# baseline (speedup 1.0000x reference)
import math

import jax
import jax.numpy as jnp
from jax.experimental import pallas as pl
from jax.experimental.pallas import tpu as pltpu

B, S = 2, 2048
HIDDEN = 768
NUM_GROUPS = 12
PER_HEAD = 12
SHORT = HIDDEN // PER_HEAD
NH = 12
HD = HIDDEN // NH
SCALEUP = 1.0 / PER_HEAD

_TS = 128
_RT = 512
_RSTEPS = S // _RT
_VH = (B * NH) // _RSTEPS


def _router_body(x_ref, w1_ref, b1_ref, w2_ref, b2_ref, ones_ref, vt_ref,
                 o_ref, vrow_ref, acc_ref):
    i = pl.program_id(0)

    @pl.when(i == 0)
    def _init():
        acc_ref[...] = jnp.zeros_like(acc_ref)

    acc_ref[...] += jnp.sum(x_ref[...], axis=1)
    vrow_ref[...] = jnp.transpose(vt_ref[...], (0, 2, 1))

    @pl.when(i == _RSTEPS - 1)
    def _finish():
        m = acc_ref[...] * (1.0 / S)
        h1 = jnp.dot(m, w1_ref[...], preferred_element_type=jnp.float32)
        h1 = h1 + b1_ref[...]
        g = 0.5 * h1 * (1.0 + jax.lax.erf(h1 * (1.0 / math.sqrt(2.0))))
        h2 = jnp.dot(g, w2_ref[...], preferred_element_type=jnp.float32)
        h2 = h2 + b2_ref[...]
        e = jnp.exp(h2 - jnp.max(h2, axis=-1, keepdims=True))
        denom = jnp.dot(e, ones_ref[...], preferred_element_type=jnp.float32)
        o_ref[...] = e / denom


_NI = S // _TS
_NBUF = 2
_NQ = 4
_HPQ = NH // _NQ


def _context_body(a_hbm, v_ref, vt_hbm, o_ref, vo_hbm, abuf, sems, psem):
    step = pl.program_id(0) * _NI + pl.program_id(1)
    nsteps = B * _NI

    def _copies(s):
        bb = s // _NI
        ii = s % _NI
        slot = jax.lax.rem(s, _NBUF)
        return [
            pltpu.make_async_copy(
                a_hbm.at[bb, pl.ds(q * _HPQ, _HPQ), pl.ds(ii * _TS, _TS), :],
                abuf.at[slot, pl.ds(q * _HPQ, _HPQ)],
                sems.at[slot, q],
            )
            for q in range(_NQ)
        ]

    @pl.when(step == 0)
    def _prologue():
        pltpu.make_async_copy(vt_hbm, vo_hbm, psem).start()
        for s in range(_NBUF):
            for c in _copies(s):
                c.start()

    for c in _copies(step):
        c.wait()

    ii = pl.program_id(1)
    slot = jax.lax.rem(step, _NBUF)
    accs = [jnp.dot(abuf[slot, h], v_ref[0, h],
                    preferred_element_type=jnp.float32)
            for h in range(NH)]
    for h in range(NH):
        o_ref[0, h, :, pl.ds(ii * _TS, _TS)] = accs[h].T * SCALEUP

    @pl.when(step + _NBUF < nsteps)
    def _prefetch():
        for c in _copies(step + _NBUF):
            c.start()

    @pl.when(step == nsteps - 1)
    def _epilogue():
        pltpu.make_async_copy(vt_hbm, vo_hbm, psem).wait()


@jax.jit
def kernel(input_data_seq, attention_probs, value_layer, W1, b1, W2, b2):
    NG = NH * NUM_GROUPS
    eye = jnp.eye(NH, dtype=jnp.float32)
    w1bd = (eye[:, None, :, None] * W1[None, :, None, :]).reshape(HIDDEN, NG)
    w2bd = (eye[:, None, :, None] * W2[None, :, None, :]).reshape(NG, NG)
    onesbd = (eye[:, None, :, None]
              * jnp.ones((NUM_GROUPS, NUM_GROUPS), jnp.float32)[None, :, None, :]
              ).reshape(NG, NG)
    b1t = jnp.tile(b1, NH).reshape(1, NG)
    b2t = jnp.tile(b2, NH).reshape(1, NG)

    vt = value_layer.transpose(0, 1, 3, 2)
    vt_flat = vt.reshape(B * NH, HD, S)

    pflat, vrow_flat = pl.pallas_call(
        _router_body,
        grid=(_RSTEPS,),
        in_specs=[
            pl.BlockSpec((B, _RT, HIDDEN), lambda i: (0, i, 0)),
            pl.BlockSpec((HIDDEN, NG), lambda i: (0, 0)),
            pl.BlockSpec((1, NG), lambda i: (0, 0)),
            pl.BlockSpec((NG, NG), lambda i: (0, 0)),
            pl.BlockSpec((1, NG), lambda i: (0, 0)),
            pl.BlockSpec((NG, NG), lambda i: (0, 0)),
            pl.BlockSpec((_VH, HD, S), lambda i: (i, 0, 0)),
        ],
        out_specs=[
            pl.BlockSpec((B, NG), lambda i: (0, 0)),
            pl.BlockSpec((_VH, S, HD), lambda i: (i, 0, 0)),
        ],
        out_shape=[
            jax.ShapeDtypeStruct((B, NG), jnp.float32),
            jax.ShapeDtypeStruct((B * NH, S, HD), jnp.float32),
        ],
        scratch_shapes=[pltpu.VMEM((B, HIDDEN), jnp.float32)],
        compiler_params=pltpu.CompilerParams(
            dimension_semantics=("arbitrary",),
        ),
    )(input_data_seq, w1bd, b1t, w2bd, b2t, onesbd, vt_flat)
    prob = pflat.reshape(B, NH, NUM_GROUPS)
    vrow = vrow_flat.reshape(B, NH, S, HD)

    grid = (B, _NI)
    ctxt, voutt = pl.pallas_call(
        _context_body,
        grid=grid,
        in_specs=[
            pl.BlockSpec(memory_space=pl.ANY),
            pl.BlockSpec((1, NH, S, HD), lambda b, i: (b, 0, 0, 0),
                         pipeline_mode=pl.Buffered(buffer_count=1)),
            pl.BlockSpec(memory_space=pl.ANY),
        ],
        out_specs=[
            pl.BlockSpec((1, NH, HD, S), lambda b, i: (b, 0, 0, 0),
                         pipeline_mode=pl.Buffered(buffer_count=1)),
            pl.BlockSpec(memory_space=pl.ANY),
        ],
        out_shape=[
            jax.ShapeDtypeStruct((B, NH, HD, S), jnp.float32),
            jax.ShapeDtypeStruct((B, NH, HD, S), jnp.float32),
        ],
        scratch_shapes=[
            pltpu.VMEM((_NBUF, NH, _TS, S), jnp.float32),
            pltpu.SemaphoreType.DMA((_NBUF, _NQ)),
            pltpu.SemaphoreType.DMA,
        ],
        compiler_params=pltpu.CompilerParams(
            dimension_semantics=("arbitrary", "arbitrary"),
        ),
    )(attention_probs, vrow, vt)

    context = ctxt.transpose(0, 3, 1, 2)
    vout = voutt.transpose(0, 1, 3, 2)
    return (prob, context, vout)


# speedup vs baseline: 2.3597x; 2.3597x over previous
import math

import jax
import jax.numpy as jnp
from jax.experimental import pallas as pl
from jax.experimental.pallas import tpu as pltpu

B, S = 2, 2048
HIDDEN = 768
NUM_GROUPS = 12
PER_HEAD = 12
SHORT = HIDDEN // PER_HEAD
NH = 12
HD = HIDDEN // NH
SCALEUP = 1.0 / PER_HEAD

_TS = 128
_RT = 512
_RSTEPS = S // _RT
_VH = (B * NH) // _RSTEPS


def _router_body(x_ref, w1_ref, b1_ref, w2_ref, b2_ref, ones_ref, vt_ref,
                 o_ref, vrow_ref, acc_ref):
    i = pl.program_id(0)

    @pl.when(i == 0)
    def _init():
        acc_ref[...] = jnp.zeros_like(acc_ref)

    acc_ref[...] += jnp.sum(x_ref[...], axis=1)
    vrow_ref[...] = jnp.transpose(vt_ref[...], (0, 2, 1))

    @pl.when(i == _RSTEPS - 1)
    def _finish():
        m = acc_ref[...] * (1.0 / S)
        h1 = jnp.dot(m, w1_ref[...], preferred_element_type=jnp.float32)
        h1 = h1 + b1_ref[...]
        g = 0.5 * h1 * (1.0 + jax.lax.erf(h1 * (1.0 / math.sqrt(2.0))))
        h2 = jnp.dot(g, w2_ref[...], preferred_element_type=jnp.float32)
        h2 = h2 + b2_ref[...]
        e = jnp.exp(h2 - jnp.max(h2, axis=-1, keepdims=True))
        denom = jnp.dot(e, ones_ref[...], preferred_element_type=jnp.float32)
        o_ref[...] = e / denom


_NI = S // _TS
_NBUF = 2
_NQ = 4
_HPQ = NH // _NQ


def _context_body(a_hbm, v_ref, o_ref, abuf, sems):
    step = pl.program_id(0) * _NI + pl.program_id(1)
    nsteps = B * _NI

    def _copies(s):
        bb = s // _NI
        ii = s % _NI
        slot = jax.lax.rem(s, _NBUF)
        return [
            pltpu.make_async_copy(
                a_hbm.at[bb, pl.ds(q * _HPQ, _HPQ), pl.ds(ii * _TS, _TS), :],
                abuf.at[slot, pl.ds(q * _HPQ, _HPQ)],
                sems.at[slot, q],
            )
            for q in range(_NQ)
        ]

    @pl.when(step == 0)
    def _prologue():
        for s in range(_NBUF):
            for c in _copies(s):
                c.start()

    for c in _copies(step):
        c.wait()

    ii = pl.program_id(1)
    slot = jax.lax.rem(step, _NBUF)
    accs = [jnp.dot(abuf[slot, h], v_ref[0, h],
                    preferred_element_type=jnp.float32)
            for h in range(NH)]
    for h in range(NH):
        o_ref[0, h, :, pl.ds(ii * _TS, _TS)] = accs[h].T * SCALEUP

    @pl.when(step + _NBUF < nsteps)
    def _prefetch():
        for c in _copies(step + _NBUF):
            c.start()


@jax.jit
def kernel(input_data_seq, attention_probs, value_layer, W1, b1, W2, b2):
    NG = NH * NUM_GROUPS
    eye = jnp.eye(NH, dtype=jnp.float32)
    w1bd = (eye[:, None, :, None] * W1[None, :, None, :]).reshape(HIDDEN, NG)
    w2bd = (eye[:, None, :, None] * W2[None, :, None, :]).reshape(NG, NG)
    onesbd = (eye[:, None, :, None]
              * jnp.ones((NUM_GROUPS, NUM_GROUPS), jnp.float32)[None, :, None, :]
              ).reshape(NG, NG)
    b1t = jnp.tile(b1, NH).reshape(1, NG)
    b2t = jnp.tile(b2, NH).reshape(1, NG)

    vt = value_layer.transpose(0, 1, 3, 2)
    vt_flat = vt.reshape(B * NH, HD, S)

    pflat, vrow_flat = pl.pallas_call(
        _router_body,
        grid=(_RSTEPS,),
        in_specs=[
            pl.BlockSpec((B, _RT, HIDDEN), lambda i: (0, i, 0)),
            pl.BlockSpec((HIDDEN, NG), lambda i: (0, 0)),
            pl.BlockSpec((1, NG), lambda i: (0, 0)),
            pl.BlockSpec((NG, NG), lambda i: (0, 0)),
            pl.BlockSpec((1, NG), lambda i: (0, 0)),
            pl.BlockSpec((NG, NG), lambda i: (0, 0)),
            pl.BlockSpec((_VH, HD, S), lambda i: (i, 0, 0)),
        ],
        out_specs=[
            pl.BlockSpec((B, NG), lambda i: (0, 0)),
            pl.BlockSpec((_VH, S, HD), lambda i: (i, 0, 0)),
        ],
        out_shape=[
            jax.ShapeDtypeStruct((B, NG), jnp.float32),
            jax.ShapeDtypeStruct((B * NH, S, HD), jnp.float32),
        ],
        scratch_shapes=[pltpu.VMEM((B, HIDDEN), jnp.float32)],
        compiler_params=pltpu.CompilerParams(
            dimension_semantics=("arbitrary",),
        ),
    )(input_data_seq, w1bd, b1t, w2bd, b2t, onesbd, vt_flat)
    prob = pflat.reshape(B, NH, NUM_GROUPS)
    vrow = vrow_flat.reshape(B, NH, S, HD)

    grid = (B, _NI)
    ctxt = pl.pallas_call(
        _context_body,
        grid=grid,
        in_specs=[
            pl.BlockSpec(memory_space=pl.ANY),
            pl.BlockSpec((1, NH, S, HD), lambda b, i: (b, 0, 0, 0),
                         pipeline_mode=pl.Buffered(buffer_count=1)),
        ],
        out_specs=pl.BlockSpec((1, NH, HD, S), lambda b, i: (b, 0, 0, 0),
                               pipeline_mode=pl.Buffered(buffer_count=1)),
        out_shape=jax.ShapeDtypeStruct((B, NH, HD, S), jnp.float32),
        scratch_shapes=[
            pltpu.VMEM((_NBUF, NH, _TS, S), jnp.float32),
            pltpu.SemaphoreType.DMA((_NBUF, _NQ)),
        ],
        compiler_params=pltpu.CompilerParams(
            dimension_semantics=("arbitrary", "arbitrary"),
        ),
    )(attention_probs, vrow)

    context = ctxt.transpose(0, 3, 1, 2)
    return (prob, context, value_layer)
